# baseline (speedup 1.0000x reference)
import jax
import jax.numpy as jnp
from jax.experimental import pallas as pl
from jax.experimental.pallas import tpu as pltpu

_ROWS = 8192
_COLS = 1024
_STREAMS = 8
_CHUNK = _ROWS // _STREAMS


def _copy_kernel(in_ref, out_ref, sems):
    for i in range(_STREAMS):
        pltpu.make_async_copy(
            in_ref.at[pl.ds(i * _CHUNK, _CHUNK), :],
            out_ref.at[pl.ds(i * _CHUNK, _CHUNK), :],
            sems.at[i],
        ).start()
    for i in range(_STREAMS):
        pltpu.make_async_copy(
            in_ref.at[pl.ds(i * _CHUNK, _CHUNK), :],
            out_ref.at[pl.ds(i * _CHUNK, _CHUNK), :],
            sems.at[i],
        ).wait()


def kernel(wpe):
    out = pl.pallas_call(
        _copy_kernel,
        out_shape=jax.ShapeDtypeStruct((_ROWS, _COLS), jnp.float32),
        in_specs=[pl.BlockSpec(memory_space=pl.ANY)],
        out_specs=pl.BlockSpec(memory_space=pl.ANY),
        scratch_shapes=[pltpu.SemaphoreType.DMA((_STREAMS,))],
    )(wpe)
    return out.reshape(1, _ROWS, _COLS)


# speedup vs baseline: 44.5642x; 44.5642x over previous
import jax
import jax.numpy as jnp
from jax.experimental import pallas as pl

_ROWS = 8192
_COLS = 1024
_BLOCK_ROWS = 1024


def _copy_kernel(in_ref, out_ref):
    out_ref[...] = in_ref[...]


def kernel(wpe):
    out = pl.pallas_call(
        _copy_kernel,
        grid=(_ROWS // _BLOCK_ROWS,),
        in_specs=[pl.BlockSpec((_BLOCK_ROWS, _COLS), lambda i: (i, 0))],
        out_specs=pl.BlockSpec((_BLOCK_ROWS, _COLS), lambda i: (i, 0)),
        out_shape=jax.ShapeDtypeStruct((_ROWS, _COLS), jnp.float32),
    )(wpe)
    return out.reshape(1, _ROWS, _COLS)
